# baseline (speedup 1.0000x reference)
import jax
import jax.numpy as jnp
from jax.experimental import pallas as pl
from jax.experimental.pallas import tpu as pltpu


def _dot_nt(a, w):
    return jax.lax.dot_general(
        a,
        w,
        dimension_numbers=(((1,), (1,)), ((), ())),
        preferred_element_type=jnp.float32,
    )


def _linear_body(a_ref, w_ref, b_ref, o_ref):
    a = a_ref[...].astype(jnp.bfloat16)
    w = w_ref[...].astype(jnp.bfloat16)
    o_ref[...] = _dot_nt(a, w) + b_ref[...]


def kernel(body_output, W, b):
    M, K = body_output.shape
    N = W.shape[0]
    BM = 1024
    b2d = b.reshape(1, N)
    return pl.pallas_call(
        _linear_body,
        grid=(M // BM,),
        in_specs=[
            pl.BlockSpec((BM, K), lambda i: (i, 0)),
            pl.BlockSpec((N, K), lambda i: (0, 0)),
            pl.BlockSpec((1, N), lambda i: (0, 0)),
        ],
        out_specs=pl.BlockSpec((BM, N), lambda i: (i, 0)),
        out_shape=jax.ShapeDtypeStruct((M, N), jnp.float32),
        compiler_params=pltpu.CompilerParams(
            dimension_semantics=("parallel",),
            vmem_limit_bytes=128 * 1024 * 1024,
        ),
    )(body_output, W, b2d)


# speedup vs baseline: 1.0773x; 1.0773x over previous
import jax
import jax.numpy as jnp
from jax.experimental import pallas as pl
from jax.experimental.pallas import tpu as pltpu

_BM = 2048
_ABUF = 3


def _dot_nt(a, w):
    return jax.lax.dot_general(
        a,
        w,
        dimension_numbers=(((1,), (1,)), ((), ())),
        preferred_element_type=jnp.float32,
    )


def _in_copy(a_hbm, a_bufs, in_sem, i):
    return pltpu.make_async_copy(
        a_hbm.at[pl.ds(i * _BM, _BM), :], a_bufs[i % _ABUF], in_sem.at[i % _ABUF]
    )


def _out_copy(o_bufs, o_hbm, out_sem, i):
    return pltpu.make_async_copy(
        o_bufs[i % 2], o_hbm.at[pl.ds(i * _BM, _BM), :], out_sem.at[i % 2]
    )


def _pipeline_body(
    a_hbm, w_hbm, b_ref, o_hbm,
    a0, a1, a2, wf, wbf, o0, o1,
    in_sem, out_sem, w_sem,
):
    n_steps = a_hbm.shape[0] // _BM
    a_bufs = [a0, a1, a2]
    o_bufs = [o0, o1]

    w_copy = pltpu.make_async_copy(w_hbm, wf, w_sem)
    w_copy.start()
    for i in range(min(_ABUF, n_steps)):
        _in_copy(a_hbm, a_bufs, in_sem, i).start()
    w_copy.wait()
    wbf[...] = wf[...].astype(jnp.bfloat16)

    for i in range(n_steps):
        _in_copy(a_hbm, a_bufs, in_sem, i).wait()
        if i >= 2:
            _out_copy(o_bufs, o_hbm, out_sem, i - 2).wait()
        ob = o_bufs[i % 2]
        ob[...] = _dot_nt(a_bufs[i % _ABUF][...].astype(jnp.bfloat16), wbf[...]) + b_ref[...]
        _out_copy(o_bufs, o_hbm, out_sem, i).start()
        if i + _ABUF < n_steps:
            _in_copy(a_hbm, a_bufs, in_sem, i + _ABUF).start()

    for i in range(max(n_steps - 2, 0), n_steps):
        _out_copy(o_bufs, o_hbm, out_sem, i).wait()


def kernel(body_output, W, b):
    M, K = body_output.shape
    N = W.shape[0]
    b2d = b.reshape(1, N)
    return pl.pallas_call(
        _pipeline_body,
        in_specs=[
            pl.BlockSpec(memory_space=pltpu.MemorySpace.HBM),
            pl.BlockSpec(memory_space=pltpu.MemorySpace.HBM),
            pl.BlockSpec(memory_space=pltpu.MemorySpace.VMEM),
        ],
        out_specs=pl.BlockSpec(memory_space=pltpu.MemorySpace.HBM),
        out_shape=jax.ShapeDtypeStruct((M, N), jnp.float32),
        scratch_shapes=[
            pltpu.VMEM((_BM, K), jnp.float32),
            pltpu.VMEM((_BM, K), jnp.float32),
            pltpu.VMEM((_BM, K), jnp.float32),
            pltpu.VMEM((N, K), jnp.float32),
            pltpu.VMEM((N, K), jnp.bfloat16),
            pltpu.VMEM((_BM, N), jnp.float32),
            pltpu.VMEM((_BM, N), jnp.float32),
            pltpu.SemaphoreType.DMA((_ABUF,)),
            pltpu.SemaphoreType.DMA((2,)),
            pltpu.SemaphoreType.DMA,
        ],
        compiler_params=pltpu.CompilerParams(
            vmem_limit_bytes=100 * 1024 * 1024,
        ),
    )(body_output, W, b2d)


# speedup vs baseline: 1.0820x; 1.0043x over previous
import jax
import jax.numpy as jnp
from jax.experimental import pallas as pl
from jax.experimental.pallas import tpu as pltpu

_BM = 2048
_ABUF = 3
_OBUF = 3


def _dot_nt(a, w):
    return jax.lax.dot_general(
        a,
        w,
        dimension_numbers=(((1,), (1,)), ((), ())),
        preferred_element_type=jnp.float32,
    )


def _in_copy(a_hbm, a_bufs, in_sem, i):
    return pltpu.make_async_copy(
        a_hbm.at[pl.ds(i * _BM, _BM), :], a_bufs[i % _ABUF], in_sem.at[i % _ABUF]
    )


def _out_copy(o_bufs, o_hbm, out_sem, i):
    return pltpu.make_async_copy(
        o_bufs[i % _OBUF], o_hbm.at[pl.ds(i * _BM, _BM), :], out_sem.at[i % _OBUF]
    )


def _pipeline_body(
    a_hbm, w_hbm, b_ref, o_hbm,
    a0, a1, a2, wf, wbf, o0, o1, o2,
    in_sem, out_sem, w_sem,
):
    n_steps = a_hbm.shape[0] // _BM
    a_bufs = [a0, a1, a2]
    o_bufs = [o0, o1, o2]

    w_copy = pltpu.make_async_copy(w_hbm, wf, w_sem)
    w_copy.start()
    for i in range(min(_ABUF, n_steps)):
        _in_copy(a_hbm, a_bufs, in_sem, i).start()
    w_copy.wait()
    wbf[...] = wf[...].astype(jnp.bfloat16)

    for i in range(n_steps):
        _in_copy(a_hbm, a_bufs, in_sem, i).wait()
        if i >= _OBUF:
            _out_copy(o_bufs, o_hbm, out_sem, i - _OBUF).wait()
        ob = o_bufs[i % _OBUF]
        ob[...] = _dot_nt(a_bufs[i % _ABUF][...].astype(jnp.bfloat16), wbf[...]) + b_ref[...]
        _out_copy(o_bufs, o_hbm, out_sem, i).start()
        if i + _ABUF < n_steps:
            _in_copy(a_hbm, a_bufs, in_sem, i + _ABUF).start()

    for i in range(max(n_steps - _OBUF, 0), n_steps):
        _out_copy(o_bufs, o_hbm, out_sem, i).wait()


def kernel(body_output, W, b):
    M, K = body_output.shape
    N = W.shape[0]
    b2d = b.reshape(1, N)
    return pl.pallas_call(
        _pipeline_body,
        in_specs=[
            pl.BlockSpec(memory_space=pltpu.MemorySpace.HBM),
            pl.BlockSpec(memory_space=pltpu.MemorySpace.HBM),
            pl.BlockSpec(memory_space=pltpu.MemorySpace.VMEM),
        ],
        out_specs=pl.BlockSpec(memory_space=pltpu.MemorySpace.HBM),
        out_shape=jax.ShapeDtypeStruct((M, N), jnp.float32),
        scratch_shapes=[
            pltpu.VMEM((_BM, K), jnp.float32),
            pltpu.VMEM((_BM, K), jnp.float32),
            pltpu.VMEM((_BM, K), jnp.float32),
            pltpu.VMEM((N, K), jnp.float32),
            pltpu.VMEM((N, K), jnp.bfloat16),
            pltpu.VMEM((_BM, N), jnp.float32),
            pltpu.VMEM((_BM, N), jnp.float32),
            pltpu.VMEM((_BM, N), jnp.float32),
            pltpu.SemaphoreType.DMA((_ABUF,)),
            pltpu.SemaphoreType.DMA((_OBUF,)),
            pltpu.SemaphoreType.DMA,
        ],
        compiler_params=pltpu.CompilerParams(
            vmem_limit_bytes=100 * 1024 * 1024,
        ),
    )(body_output, W, b2d)


# speedup vs baseline: 1.0955x; 1.0125x over previous
import jax
import jax.numpy as jnp
from jax.experimental import pallas as pl
from jax.experimental.pallas import tpu as pltpu

_BM = 1024
_ABUF = 4
_OBUF = 4


def _dot_nt(a, w):
    return jax.lax.dot_general(
        a,
        w,
        dimension_numbers=(((1,), (1,)), ((), ())),
        preferred_element_type=jnp.float32,
    )


def _in_copy(a_hbm, a_bufs, in_sem, i):
    return pltpu.make_async_copy(
        a_hbm.at[pl.ds(i * _BM, _BM), :], a_bufs[i % _ABUF], in_sem.at[i % _ABUF]
    )


def _out_copy(o_bufs, o_hbm, out_sem, i):
    return pltpu.make_async_copy(
        o_bufs[i % _OBUF], o_hbm.at[pl.ds(i * _BM, _BM), :], out_sem.at[i % _OBUF]
    )


def _pipeline_body(
    a_hbm, w_hbm, b_ref, o_hbm,
    a0, a1, a2, a3, wf, wbf, o0, o1, o2, o3,
    in_sem, out_sem, w_sem,
):
    n_steps = a_hbm.shape[0] // _BM
    a_bufs = [a0, a1, a2, a3]
    o_bufs = [o0, o1, o2, o3]

    w_copy = pltpu.make_async_copy(w_hbm, wf, w_sem)
    w_copy.start()
    for i in range(min(_ABUF, n_steps)):
        _in_copy(a_hbm, a_bufs, in_sem, i).start()
    w_copy.wait()
    wbf[...] = wf[...].astype(jnp.bfloat16)

    for i in range(n_steps):
        _in_copy(a_hbm, a_bufs, in_sem, i).wait()
        if i >= _OBUF:
            _out_copy(o_bufs, o_hbm, out_sem, i - _OBUF).wait()
        ob = o_bufs[i % _OBUF]
        ob[...] = _dot_nt(a_bufs[i % _ABUF][...].astype(jnp.bfloat16), wbf[...]) + b_ref[...]
        _out_copy(o_bufs, o_hbm, out_sem, i).start()
        if i + _ABUF < n_steps:
            _in_copy(a_hbm, a_bufs, in_sem, i + _ABUF).start()

    for i in range(max(n_steps - _OBUF, 0), n_steps):
        _out_copy(o_bufs, o_hbm, out_sem, i).wait()


def kernel(body_output, W, b):
    M, K = body_output.shape
    N = W.shape[0]
    b2d = b.reshape(1, N)
    return pl.pallas_call(
        _pipeline_body,
        in_specs=[
            pl.BlockSpec(memory_space=pltpu.MemorySpace.HBM),
            pl.BlockSpec(memory_space=pltpu.MemorySpace.HBM),
            pl.BlockSpec(memory_space=pltpu.MemorySpace.VMEM),
        ],
        out_specs=pl.BlockSpec(memory_space=pltpu.MemorySpace.HBM),
        out_shape=jax.ShapeDtypeStruct((M, N), jnp.float32),
        scratch_shapes=[
            pltpu.VMEM((_BM, K), jnp.float32),
            pltpu.VMEM((_BM, K), jnp.float32),
            pltpu.VMEM((_BM, K), jnp.float32),
            pltpu.VMEM((_BM, K), jnp.float32),
            pltpu.VMEM((N, K), jnp.float32),
            pltpu.VMEM((N, K), jnp.bfloat16),
            pltpu.VMEM((_BM, N), jnp.float32),
            pltpu.VMEM((_BM, N), jnp.float32),
            pltpu.VMEM((_BM, N), jnp.float32),
            pltpu.VMEM((_BM, N), jnp.float32),
            pltpu.SemaphoreType.DMA((_ABUF,)),
            pltpu.SemaphoreType.DMA((_OBUF,)),
            pltpu.SemaphoreType.DMA,
        ],
        compiler_params=pltpu.CompilerParams(
            vmem_limit_bytes=100 * 1024 * 1024,
        ),
    )(body_output, W, b2d)
